# baseline (speedup 1.0000x reference)
import functools

import jax
import jax.numpy as jnp
from jax import lax
from jax.experimental import pallas as pl
from jax.experimental.pallas import tpu as pltpu
from jax.experimental.pallas import tpu_sc as plsc

NUM_NODES = 2048
MAX_VISIT = 20
EMB = 128
HID = 256
OUT = 128
N_LAYERS = 3
N = 25600
E = 409600
B = 256
DECAY = 0.03


def _mm_body(a_ref, w_ref, b_ref, o_ref, *, relu, bf16=False):
    a = a_ref[...]
    w = w_ref[...]
    if bf16:
        a = a.astype(jnp.bfloat16)
        w = w.astype(jnp.bfloat16)
    acc = jnp.dot(a, w, preferred_element_type=jnp.float32)
    acc = acc + b_ref[...]
    if relu:
        acc = jnp.maximum(acc, 0.0)
    o_ref[...] = acc


def _mm(a, w, b, relu=False, bm=256, bo=None, bf16=False):
    M, K = a.shape
    K2, O = w.shape
    assert K == K2, (a.shape, w.shape)
    if bo is None:
        bo = O
    assert M % bm == 0 and O % bo == 0, (M, O, bm, bo)
    b2 = b.reshape(1, O)
    grid = (M // bm, O // bo)
    return pl.pallas_call(
        functools.partial(_mm_body, relu=relu, bf16=bf16),
        grid=grid,
        in_specs=[
            pl.BlockSpec((bm, K), lambda i, j: (i, 0)),
            pl.BlockSpec((K, bo), lambda i, j: (0, j)),
            pl.BlockSpec((1, bo), lambda i, j: (0, j)),
        ],
        out_specs=pl.BlockSpec((bm, bo), lambda i, j: (i, j)),
        out_shape=jax.ShapeDtypeStruct((M, O), jnp.float32),
    )(a, w, b2)


NB = 320
BKT = N // NB
CK = 128
NWORK = 32
BPW = NB // NWORK


def _sc_mp_body(xa, t_hbm, srcs, rd, bounds, zeros, out,
                t_v, agg, xrows, sidx, rd_v, bnd_v, sem):
    c = lax.axis_index("c")
    s = lax.axis_index("s")
    w = s * 2 + c
    pltpu.sync_copy(t_hbm, t_v)
    pltpu.sync_copy(bounds, bnd_v)
    for k in range(BPW):
        b = w * BPW + k
        base = b * BKT
        pltpu.sync_copy(zeros, agg)
        bv = bnd_v[pl.ds(b, 16)]
        lo = bv[0]
        hi = bv[1]
        alo = (lo // CK) * CK
        nc = jnp.maximum((hi - alo + CK - 1) // CK, 0)

        def chunk_body(m, carry):
            start = alo + m * CK
            pltpu.sync_copy(srcs.at[pl.ds(start, CK)], sidx)
            pltpu.sync_copy(rd.at[pl.ds(2 * start, 2 * CK)],
                            rd_v.at[pl.ds(0, 2 * CK)])
            pltpu.async_copy(xa.at[sidx], xrows, sem).wait()

            def edge_body(jj, carry2):
                e = start + jj
                v = rd_v[pl.ds(2 * jj, 16)]
                r = v[0]
                dl = v[1] - base

                @pl.when((e >= lo) & (e < hi))
                def _():
                    for si in range(HID // 16):
                        xv = xrows[jj, pl.ds(si * 16, 16)]
                        tv = t_v[r, pl.ds(si * 16, 16)]
                        mv = jnp.maximum(xv + tv, 0.0)
                        plsc.addupdate(agg.at[dl, pl.ds(si * 16, 16)], mv)

                return carry2

            lax.fori_loop(0, CK, edge_body, 0)
            return carry

        lax.fori_loop(0, nc, chunk_body, 0)
        pltpu.sync_copy(agg, out.at[pl.ds(base, BKT)])


_SC_MP = None


def _sc_mp():
    global _SC_MP
    if _SC_MP is None:
        mesh = plsc.VectorSubcoreMesh(core_axis_name="c", subcore_axis_name="s")
        _SC_MP = pl.kernel(
            _sc_mp_body, mesh=mesh,
            out_type=jax.ShapeDtypeStruct((N, HID), jnp.float32),
            scratch_types=[
                pltpu.VMEM((256, HID), jnp.float32),
                pltpu.VMEM((BKT, HID), jnp.float32),
                pltpu.VMEM((CK, HID), jnp.float32),
                pltpu.VMEM((CK,), jnp.int32),
                pltpu.VMEM((2 * CK + 16,), jnp.int32),
                pltpu.VMEM((336,), jnp.int32),
                pltpu.SemaphoreType.DMA,
            ])
    return _SC_MP


def kernel(node_ids, rel_ids, edge_index, batch, visit_node, ehr_nodes,
           node_emb, rel_emb, lin_W, lin_b, alpha_W, alpha_b, beta_W, beta_b,
           conv_W, conv_b, WR_W, WR_b, gate, importance, mlp_W, mlp_b):
    V = MAX_VISIT
    j = jnp.arange(V, dtype=jnp.float32)
    lam = jnp.exp(DECAY * (V - j)).reshape(1, V)

    src = edge_index[0]
    dst = edge_index[1]

    zk = _mm(node_emb, lin_W.T, jnp.zeros((HID,), jnp.float32))
    z = zk + lin_b
    x = z[node_ids]

    T = _mm(rel_emb, lin_W.T, lin_b)
    Ts = []
    for l in range(N_LAYERS):
        w_rel = T @ WR_W[l].T + WR_b[l]
        T = w_rel * T
        Ts.append(T)

    vn2 = visit_node.reshape(B * V, NUM_NODES)
    aw = jnp.concatenate(
        [alpha_W[l].T for l in range(N_LAYERS)]
        + [jnp.transpose(beta_W, (2, 0, 1)).reshape(NUM_NODES, N_LAYERS)]
        + [jnp.zeros((NUM_NODES, 512 - N_LAYERS), jnp.float32)], axis=1)
    ab = jnp.concatenate(
        [alpha_b.reshape(-1), beta_b.reshape(-1),
         jnp.zeros((512 - N_LAYERS,), jnp.float32)])
    logits = _mm(vn2, aw, ab, bm=320, bo=512, bf16=True)

    attns = []
    for l in range(N_LAYERS):
        al = logits[:, l * NUM_NODES:(l + 1) * NUM_NODES].reshape(B, V, NUM_NODES)
        alpha = jax.nn.softmax(al, axis=1)
        bl = logits[:, N_LAYERS * NUM_NODES + l].reshape(B, V)
        beta = jnp.tanh(bl) * lam
        attns.append(jnp.einsum('bvn,bv->bn', alpha, beta))

    pres = jnp.zeros((N,), jnp.bool_).at[src].set(True)
    big = jnp.float32(jnp.inf)
    a_layers = []
    for l in range(N_LAYERS):
        a_attn = attns[l][batch, node_ids]
        impn = importance[l][node_ids]
        mn = jnp.min(jnp.where(pres, impn, big))
        mx = jnp.max(jnp.where(pres, impn, -big))
        imp_norm = (impn - mn) / (mx - mn + 1e-08)
        g = jax.nn.sigmoid(gate[l])
        a_layers.append(a_attn * (1.0 + g * (imp_norm - 1.0)))

    sdst, ssrc, srel = jax.lax.sort(
        (dst.astype(jnp.int32), src.astype(jnp.int32),
         rel_ids.astype(jnp.int32)), num_keys=1)
    bnds = jnp.searchsorted(
        sdst, jnp.arange(NB + 1, dtype=jnp.int32) * BKT,
        side='left').astype(jnp.int32)
    bnds = jnp.concatenate([bnds, jnp.full((336 - NB - 1,), E, jnp.int32)])
    pad = jnp.zeros((CK,), jnp.int32)
    ssrc = jnp.concatenate([ssrc, pad])
    srel = jnp.concatenate([srel, pad])
    sdst = jnp.concatenate([sdst, pad])
    rd = jnp.stack([srel, sdst], axis=1).reshape(-1)
    zeros_hbm = jnp.zeros((BKT, HID), jnp.float32)
    for l in range(N_LAYERS):
        xa = x * a_layers[l][:, None]
        agg = _sc_mp()(xa, Ts[l], ssrc, rd, bnds, zeros_hbm)
        x = _mm(agg + x, conv_W[l].T, conv_b[l], relu=True, bm=512)

    ones = jnp.ones((N,), jnp.float32)
    counts = jax.ops.segment_sum(ones, batch, num_segments=B)
    x_graph = jax.ops.segment_sum(x, batch, num_segments=B) \
        / jnp.maximum(counts, 1.0)[:, None]
    s = jnp.sum(ehr_nodes, axis=1, keepdims=True)
    x_node = _mm(ehr_nodes, zk, jnp.zeros((HID,), jnp.float32)) / s + lin_b
    feats = jnp.concatenate([x_graph, x_node], axis=1)
    logits_out = _mm(feats, mlp_W.T, mlp_b)
    return logits_out


# speedup vs baseline: 1.0095x; 1.0095x over previous
import functools

import jax
import jax.numpy as jnp
from jax import lax
from jax.experimental import pallas as pl
from jax.experimental.pallas import tpu as pltpu
from jax.experimental.pallas import tpu_sc as plsc

NUM_NODES = 2048
MAX_VISIT = 20
EMB = 128
HID = 256
OUT = 128
N_LAYERS = 3
N = 25600
E = 409600
B = 256
DECAY = 0.03


def _mm_body(a_ref, w_ref, b_ref, o_ref, *, relu, bf16=False):
    a = a_ref[...]
    w = w_ref[...]
    if bf16:
        a = a.astype(jnp.bfloat16)
        w = w.astype(jnp.bfloat16)
    acc = jnp.dot(a, w, preferred_element_type=jnp.float32)
    acc = acc + b_ref[...]
    if relu:
        acc = jnp.maximum(acc, 0.0)
    o_ref[...] = acc


def _mm(a, w, b, relu=False, bm=256, bo=None, bf16=False):
    M, K = a.shape
    K2, O = w.shape
    assert K == K2, (a.shape, w.shape)
    if bo is None:
        bo = O
    assert M % bm == 0 and O % bo == 0, (M, O, bm, bo)
    b2 = b.reshape(1, O)
    grid = (M // bm, O // bo)
    return pl.pallas_call(
        functools.partial(_mm_body, relu=relu, bf16=bf16),
        grid=grid,
        in_specs=[
            pl.BlockSpec((bm, K), lambda i, j: (i, 0)),
            pl.BlockSpec((K, bo), lambda i, j: (0, j)),
            pl.BlockSpec((1, bo), lambda i, j: (0, j)),
        ],
        out_specs=pl.BlockSpec((bm, bo), lambda i, j: (i, j)),
        out_shape=jax.ShapeDtypeStruct((M, O), jnp.float32),
    )(a, w, b2)


NB = 320
BKT = N // NB
CK = 128
NWORK = 32
BPW = NB // NWORK


def _sc_mp_body(xa, t_hbm, srcs, rd, bounds, zeros, out,
                t_v, agg, xrows, sidx, rd_v, bnd_v, sem):
    c = lax.axis_index("c")
    s = lax.axis_index("s")
    w = s * 2 + c
    pltpu.sync_copy(t_hbm, t_v)
    pltpu.sync_copy(bounds, bnd_v)
    for k in range(BPW):
        b = w * BPW + k
        base = b * BKT
        pltpu.sync_copy(zeros, agg)
        bv = bnd_v[pl.ds(b, 16)]
        lo = bv[0]
        hi = bv[1]
        alo = (lo // CK) * CK
        nc = jnp.maximum((hi - alo + CK - 1) // CK, 0)

        def chunk_body(m, carry):
            start = alo + m * CK
            pltpu.sync_copy(srcs.at[pl.ds(start, CK)], sidx)
            pltpu.sync_copy(rd.at[pl.ds(2 * start, 2 * CK)],
                            rd_v.at[pl.ds(0, 2 * CK)])
            pltpu.async_copy(xa.at[sidx], xrows, sem).wait()

            def edge_body(jj, carry2):
                e = start + jj
                v = rd_v[pl.ds(2 * jj, 16)]
                r = v[0]
                dl = v[1] - base

                @pl.when((e >= lo) & (e < hi))
                def _():
                    for si in range(HID // 16):
                        xv = xrows[jj, pl.ds(si * 16, 16)]
                        tv = t_v[r, pl.ds(si * 16, 16)]
                        mv = jnp.maximum(xv + tv, 0.0)
                        plsc.addupdate(agg.at[dl, pl.ds(si * 16, 16)], mv)

                return carry2

            lax.fori_loop(0, CK, edge_body, 0, unroll=4)
            return carry

        lax.fori_loop(0, nc, chunk_body, 0)
        pltpu.sync_copy(agg, out.at[pl.ds(base, BKT)])


_SC_MP = None


def _sc_mp():
    global _SC_MP
    if _SC_MP is None:
        mesh = plsc.VectorSubcoreMesh(core_axis_name="c", subcore_axis_name="s")
        _SC_MP = pl.kernel(
            _sc_mp_body, mesh=mesh,
            out_type=jax.ShapeDtypeStruct((N, HID), jnp.float32),
            scratch_types=[
                pltpu.VMEM((256, HID), jnp.float32),
                pltpu.VMEM((BKT, HID), jnp.float32),
                pltpu.VMEM((CK, HID), jnp.float32),
                pltpu.VMEM((CK,), jnp.int32),
                pltpu.VMEM((2 * CK + 16,), jnp.int32),
                pltpu.VMEM((336,), jnp.int32),
                pltpu.SemaphoreType.DMA,
            ])
    return _SC_MP


def kernel(node_ids, rel_ids, edge_index, batch, visit_node, ehr_nodes,
           node_emb, rel_emb, lin_W, lin_b, alpha_W, alpha_b, beta_W, beta_b,
           conv_W, conv_b, WR_W, WR_b, gate, importance, mlp_W, mlp_b):
    V = MAX_VISIT
    j = jnp.arange(V, dtype=jnp.float32)
    lam = jnp.exp(DECAY * (V - j)).reshape(1, V)

    src = edge_index[0]
    dst = edge_index[1]

    zk = _mm(node_emb, lin_W.T, jnp.zeros((HID,), jnp.float32))
    z = zk + lin_b
    x = z[node_ids]

    T = _mm(rel_emb, lin_W.T, lin_b)
    Ts = []
    for l in range(N_LAYERS):
        w_rel = T @ WR_W[l].T + WR_b[l]
        T = w_rel * T
        Ts.append(T)

    vn2 = visit_node.reshape(B * V, NUM_NODES)
    aw = jnp.concatenate(
        [alpha_W[l].T for l in range(N_LAYERS)]
        + [jnp.transpose(beta_W, (2, 0, 1)).reshape(NUM_NODES, N_LAYERS)]
        + [jnp.zeros((NUM_NODES, 512 - N_LAYERS), jnp.float32)], axis=1)
    ab = jnp.concatenate(
        [alpha_b.reshape(-1), beta_b.reshape(-1),
         jnp.zeros((512 - N_LAYERS,), jnp.float32)])
    logits = _mm(vn2, aw, ab, bm=320, bo=512, bf16=True)

    attns = []
    for l in range(N_LAYERS):
        al = logits[:, l * NUM_NODES:(l + 1) * NUM_NODES].reshape(B, V, NUM_NODES)
        alpha = jax.nn.softmax(al, axis=1)
        bl = logits[:, N_LAYERS * NUM_NODES + l].reshape(B, V)
        beta = jnp.tanh(bl) * lam
        attns.append(jnp.einsum('bvn,bv->bn', alpha, beta))

    pres = jnp.zeros((N,), jnp.bool_).at[src].set(True)
    big = jnp.float32(jnp.inf)
    a_layers = []
    for l in range(N_LAYERS):
        a_attn = attns[l][batch, node_ids]
        impn = importance[l][node_ids]
        mn = jnp.min(jnp.where(pres, impn, big))
        mx = jnp.max(jnp.where(pres, impn, -big))
        imp_norm = (impn - mn) / (mx - mn + 1e-08)
        g = jax.nn.sigmoid(gate[l])
        a_layers.append(a_attn * (1.0 + g * (imp_norm - 1.0)))

    sdst, ssrc, srel = jax.lax.sort(
        (dst.astype(jnp.int32), src.astype(jnp.int32),
         rel_ids.astype(jnp.int32)), num_keys=1)
    bnds = jnp.searchsorted(
        sdst, jnp.arange(NB + 1, dtype=jnp.int32) * BKT,
        side='left').astype(jnp.int32)
    bnds = jnp.concatenate([bnds, jnp.full((336 - NB - 1,), E, jnp.int32)])
    pad = jnp.zeros((CK,), jnp.int32)
    ssrc = jnp.concatenate([ssrc, pad])
    srel = jnp.concatenate([srel, pad])
    sdst = jnp.concatenate([sdst, pad])
    rd = jnp.stack([srel, sdst], axis=1).reshape(-1)
    zeros_hbm = jnp.zeros((BKT, HID), jnp.float32)
    for l in range(N_LAYERS):
        xa = x * a_layers[l][:, None]
        agg = _sc_mp()(xa, Ts[l], ssrc, rd, bnds, zeros_hbm)
        x = _mm(agg + x, conv_W[l].T, conv_b[l], relu=True, bm=512)

    ones = jnp.ones((N,), jnp.float32)
    counts = jax.ops.segment_sum(ones, batch, num_segments=B)
    x_graph = jax.ops.segment_sum(x, batch, num_segments=B) \
        / jnp.maximum(counts, 1.0)[:, None]
    s = jnp.sum(ehr_nodes, axis=1, keepdims=True)
    x_node = _mm(ehr_nodes, zk, jnp.zeros((HID,), jnp.float32)) / s + lin_b
    feats = jnp.concatenate([x_graph, x_node], axis=1)
    logits_out = _mm(feats, mlp_W.T, mlp_b)
    return logits_out
